# baseline (speedup 1.0000x reference)
import functools

import jax
import jax.numpy as jnp
from jax import lax
from jax.experimental import pallas as pl
from jax.experimental.pallas import tpu as pltpu
from jax.experimental.pallas import tpu_sc as plsc

B, P, E = 64, 1024, 768
B_TC = 36
B_SC = B - B_TC
NC, NS, L = 2, 16, 16
NW = NC * NS
ROWS = P // NW
LANES_PER_ROW = E // L


def _tc_add(p_ref, t_ref, o_ref):
    o_ref[...] = p_ref[...] + t_ref[...]


def _tc_kernel(patches, table):
    return pl.pallas_call(
        _tc_add,
        grid=(B_TC,),
        in_specs=[
            pl.BlockSpec((1, P, E), lambda b: (b, 0, 0)),
            pl.BlockSpec((P, E), lambda b: (0, 0)),
        ],
        out_specs=pl.BlockSpec((1, P, E), lambda b: (b, 0, 0)),
        out_shape=jax.ShapeDtypeStruct((B_TC, P, E), patches.dtype),
    )(patches, table)


def _sc_add(patches_hbm, table_hbm, out_hbm, tab_v, ibufs, obufs, isems, osems):
    wid = lax.axis_index("s") * NC + lax.axis_index("c")
    p0 = wid * ROWS
    rows = pl.ds(p0, ROWS)
    pltpu.sync_copy(table_hbm.at[rows], tab_v)

    pltpu.make_async_copy(patches_hbm.at[B_TC, rows], ibufs[0], isems[0]).start()
    pltpu.make_async_copy(patches_hbm.at[B_TC + 1, rows], ibufs[1], isems[1]).start()

    def pair_body(i, _):
        for q in range(2):
            ib, ob, si, so = ibufs[q], obufs[q], isems[q], osems[q]
            b = 2 * i + q
            pltpu.make_async_copy(patches_hbm.at[B_TC + b, rows], ib, si).wait()

            @pl.when(i > 0)
            def _drain():
                pltpu.make_async_copy(ob, out_hbm.at[b, rows], so).wait()

            @plsc.parallel_loop(0, ROWS)
            def row_body(r):
                for j in range(LANES_PER_ROW):
                    sl = pl.ds(j * L, L)
                    ob[r, sl] = ib[r, sl] + tab_v[r, sl]

            pltpu.make_async_copy(ob, out_hbm.at[b, rows], so).start()

            @pl.when(b + 2 < B_SC)
            def _prefetch():
                pltpu.make_async_copy(
                    patches_hbm.at[B_TC + b + 2, rows], ib, si).start()

        return 0

    lax.fori_loop(0, B_SC // 2, pair_body, 0)
    pltpu.make_async_copy(obufs[0], out_hbm.at[B_SC - 2, rows], osems[0]).wait()
    pltpu.make_async_copy(obufs[1], out_hbm.at[B_SC - 1, rows], osems[1]).wait()


def _sc_body(patches_hbm, table_hbm, out_hbm, tab_v,
             ibuf0, ibuf1, obuf0, obuf1, isem0, isem1, osem0, osem1):
    _sc_add(patches_hbm, table_hbm, out_hbm, tab_v,
            (ibuf0, ibuf1), (obuf0, obuf1), (isem0, isem1), (osem0, osem1))


_sc_kernel = functools.partial(
    pl.kernel,
    out_type=jax.ShapeDtypeStruct((B_SC, P, E), jnp.float32),
    mesh=plsc.VectorSubcoreMesh(core_axis_name="c", subcore_axis_name="s"),
    scratch_types=[
        pltpu.VMEM((ROWS, E), jnp.float32),
        pltpu.VMEM((ROWS, E), jnp.float32),
        pltpu.VMEM((ROWS, E), jnp.float32),
        pltpu.VMEM((ROWS, E), jnp.float32),
        pltpu.VMEM((ROWS, E), jnp.float32),
        pltpu.SemaphoreType.DMA,
        pltpu.SemaphoreType.DMA,
        pltpu.SemaphoreType.DMA,
        pltpu.SemaphoreType.DMA,
    ],
)(_sc_body)


def kernel(patches, table):
    tc_out = _tc_kernel(patches, table)
    sc_out = _sc_kernel(patches, table)
    return jnp.concatenate([tc_out, sc_out], axis=0)


# speedup vs baseline: 1.7437x; 1.7437x over previous
import jax
import jax.numpy as jnp
from jax.experimental import pallas as pl


def _add_kernel(p_ref, t_ref, o_ref):
    o_ref[...] = p_ref[...] + t_ref[...]


def kernel(patches, table):
    B, P, E = patches.shape
    BP = 512
    return pl.pallas_call(
        _add_kernel,
        grid=(P // BP, B),
        in_specs=[
            pl.BlockSpec((1, BP, E), lambda p, b: (b, p, 0)),
            pl.BlockSpec((BP, E), lambda p, b: (p, 0)),
        ],
        out_specs=pl.BlockSpec((1, BP, E), lambda p, b: (b, p, 0)),
        out_shape=jax.ShapeDtypeStruct((B, P, E), patches.dtype),
    )(patches, table)


# speedup vs baseline: 2.1728x; 1.2461x over previous
import jax
import jax.numpy as jnp
from jax.experimental import pallas as pl


def _add_kernel(p_ref, t_ref, o_ref):
    o_ref[...] = p_ref[...] + t_ref[...]


def kernel(patches, table):
    B, P, E = patches.shape
    BB = 2
    return pl.pallas_call(
        _add_kernel,
        grid=(B // BB,),
        in_specs=[
            pl.BlockSpec((BB, P, E), lambda b: (b, 0, 0)),
            pl.BlockSpec((P, E), lambda b: (0, 0)),
        ],
        out_specs=pl.BlockSpec((BB, P, E), lambda b: (b, 0, 0)),
        out_shape=jax.ShapeDtypeStruct((B, P, E), patches.dtype),
    )(patches, table)


# speedup vs baseline: 2.1960x; 1.0107x over previous
import jax
import jax.numpy as jnp
from jax.experimental import pallas as pl


def _add_kernel(p_ref, t_ref, o_ref):
    o_ref[...] = p_ref[...] + t_ref[...]


def kernel(patches, table):
    B, P, E = patches.shape
    BB = 4
    return pl.pallas_call(
        _add_kernel,
        grid=(B // BB,),
        in_specs=[
            pl.BlockSpec((BB, P, E), lambda b: (b, 0, 0)),
            pl.BlockSpec((P, E), lambda b: (0, 0)),
        ],
        out_specs=pl.BlockSpec((BB, P, E), lambda b: (b, 0, 0)),
        out_shape=jax.ShapeDtypeStruct((B, P, E), patches.dtype),
    )(patches, table)
